# Initial kernel scaffold; baseline (speedup 1.0000x reference)
import jax
import jax.numpy as jnp
from jax.experimental import pallas as pl


def kernel(x):
    raise NotImplementedError("write your pallas kernel here")


# speedup vs baseline: 1.0862x; 1.0862x over previous
import jax
import jax.numpy as jnp
from jax.experimental import pallas as pl
from jax.experimental.pallas import tpu as pltpu

_PERCENTAGE = 0.5
_SEED = 0

_ROWS_PER_BLOCK = 16


def _masked_relu_body(mask_ref, x_ref, o_ref):
    b = pl.program_id(0)
    sel = mask_ref[b] != 0

    @pl.when(sel)
    def _():
        o_ref[...] = jnp.maximum(x_ref[...], 0.0)

    @pl.when(jnp.logical_not(sel))
    def _():
        o_ref[...] = x_ref[...]


def kernel(x):
    B = x.shape[0]
    subset_size = int(B * _PERCENTAGE)
    perm = jax.random.permutation(jax.random.key(_SEED), B)
    idx = perm[:subset_size]
    mask = jnp.zeros((B,), jnp.int32).at[idx].set(1)

    R = x.shape[1]
    C = x.shape[2] * x.shape[3]
    xv = x.reshape(B, R, C)

    grid = (B, R // _ROWS_PER_BLOCK)
    out = pl.pallas_call(
        _masked_relu_body,
        grid_spec=pltpu.PrefetchScalarGridSpec(
            num_scalar_prefetch=1,
            grid=grid,
            in_specs=[
                pl.BlockSpec((1, _ROWS_PER_BLOCK, C), lambda b, r, m: (b, r, 0)),
            ],
            out_specs=pl.BlockSpec((1, _ROWS_PER_BLOCK, C), lambda b, r, m: (b, r, 0)),
        ),
        out_shape=jax.ShapeDtypeStruct((B, R, C), x.dtype),
    )(mask, xv)
    return out.reshape(x.shape)
